# baseline (speedup 1.0000x reference)
import numpy as np
import jax
import jax.numpy as jnp
from jax.experimental import pallas as pl

_N = 21
_HAND_EDGES = [[0, 1], [1, 2], [2, 3], [3, 4], [0, 5], [5, 6], [6, 7], [7, 8],
               [0, 9], [9, 10], [10, 11], [11, 12], [0, 13], [13, 14],
               [14, 15], [15, 16], [0, 17], [17, 18], [18, 19], [19, 20]]

_CLOSED = []
for n in range(_N):
    nb = {n}
    for i, j in _HAND_EDGES:
        if i == n:
            nb.add(j)
        if j == n:
            nb.add(i)
    _CLOSED.append(sorted(nb))
_INVDEG = [1.0 / len(c) for c in _CLOSED]


def _leaky(v):
    return jnp.maximum(v, 0.01 * v)


def _gcn_body(x_ref, w1_ref, b1_ref, w2_ref, b2_ref, w3_ref, b3_ref, o_ref):
    bb = x_ref.shape[1]
    w1 = w1_ref[:]
    w2 = w2_ref[:]
    w3 = w3_ref[:]
    b1 = b1_ref[:]
    b2 = b2_ref[:]
    b3 = b3_ref[:]

    def layer(h2d, w, b, out_w, act):
        y = jnp.dot(h2d.astype(jnp.bfloat16), w,
                    preferred_element_type=jnp.float32)
        y3 = y.reshape(_N, bb, 2 * out_w)
        y0 = y3[:, :, :out_w]
        y1 = y3[:, :, out_w:]
        outs = []
        for n in range(_N):
            s = None
            for m in _CLOSED[n]:
                s = y1[m] if s is None else s + y1[m]
            acc = (y0[n] + b) + (y1[n] - _INVDEG[n] * s)
            outs.append(_leaky(acc) if act else acc)
        return outs

    x2 = x_ref[:].reshape(_N * bb, 256)
    hs = layer(x2, w1, b1, 256, True)
    hs = layer(jnp.concatenate(hs, axis=0), w2, b2, 64, True)
    os_ = layer(jnp.concatenate(hs, axis=0), w3, b3, 3, False)
    o_ref[:] = jnp.concatenate(os_, axis=1)


def kernel(x, W1, b1, W2, b2, W3, b3):
    B = x.shape[0]
    Bb = 128
    xt = jnp.transpose(x, (1, 0, 2))
    wc1 = jnp.concatenate([W1[0], W1[1]], axis=1).astype(jnp.bfloat16)
    wc2 = jnp.concatenate([W2[0], W2[1]], axis=1).astype(jnp.bfloat16)
    wc3 = jnp.concatenate([W3[0], W3[1]], axis=1).astype(jnp.bfloat16)
    out2 = pl.pallas_call(
        _gcn_body,
        grid=(B // Bb,),
        in_specs=[
            pl.BlockSpec((_N, Bb, 256), lambda i: (0, i, 0)),
            pl.BlockSpec((256, 512), lambda i: (0, 0)),
            pl.BlockSpec((1, 256), lambda i: (0, 0)),
            pl.BlockSpec((256, 128), lambda i: (0, 0)),
            pl.BlockSpec((1, 64), lambda i: (0, 0)),
            pl.BlockSpec((64, 6), lambda i: (0, 0)),
            pl.BlockSpec((1, 3), lambda i: (0, 0)),
        ],
        out_specs=pl.BlockSpec((Bb, _N * 3), lambda i: (i, 0)),
        out_shape=jax.ShapeDtypeStruct((B, _N * 3), jnp.float32),
    )(xt, wc1, b1.reshape(1, 256), wc2, b2.reshape(1, 64),
      wc3, b3.reshape(1, 3))
    return out2.reshape(B, _N, 3)


# speedup vs baseline: 1.0705x; 1.0705x over previous
import numpy as np
import jax
import jax.numpy as jnp
from jax.experimental import pallas as pl

_N = 21
_HAND_EDGES = [[0, 1], [1, 2], [2, 3], [3, 4], [0, 5], [5, 6], [6, 7], [7, 8],
               [0, 9], [9, 10], [10, 11], [11, 12], [0, 13], [13, 14],
               [14, 15], [15, 16], [0, 17], [17, 18], [18, 19], [19, 20]]

_CLOSED = []
for n in range(_N):
    nb = {n}
    for i, j in _HAND_EDGES:
        if i == n:
            nb.add(j)
        if j == n:
            nb.add(i)
    _CLOSED.append(sorted(nb))
_INVDEG = [1.0 / len(c) for c in _CLOSED]


def _leaky(v):
    return jnp.maximum(v, 0.01 * v)


def _gcn_body(x_ref, w1_ref, b1_ref, w2_ref, b2_ref, w3_ref, b3_ref, o_ref):
    bb = x_ref.shape[1]
    w1 = w1_ref[:]
    w2 = w2_ref[:]
    w3 = w3_ref[:]
    b1 = b1_ref[:]
    b2 = b2_ref[:]
    b3 = b3_ref[:]

    def layer(h2d, w, b, out_w, act):
        y = jnp.dot(h2d, w, preferred_element_type=jnp.float32)
        y3 = y.reshape(_N, bb, 2 * out_w)
        ya = y3[:, :, :out_w]
        y1 = y3[:, :, out_w:]
        outs = []
        for n in range(_N):
            s = None
            for m in _CLOSED[n]:
                s = y1[m] if s is None else s + y1[m]
            acc = (ya[n] + b) - _INVDEG[n] * s
            outs.append(_leaky(acc) if act else acc)
        return outs

    x2 = x_ref[:].reshape(_N * bb, 256)
    hs = layer(x2, w1, b1, 256, True)
    hs = layer(jnp.concatenate(hs, axis=0), w2, b2, 64, True)
    os_ = layer(jnp.concatenate(hs, axis=0), w3, b3, 3, False)
    o_ref[:] = jnp.concatenate(os_, axis=1)


def kernel(x, W1, b1, W2, b2, W3, b3):
    B = x.shape[0]
    Bb = 256
    xt = jnp.transpose(x, (1, 0, 2))
    wc1 = jnp.concatenate([W1[0] + W1[1], W1[1]], axis=1)
    wc2 = jnp.concatenate([W2[0] + W2[1], W2[1]], axis=1)
    wc3 = jnp.concatenate([W3[0] + W3[1], W3[1]], axis=1)
    out2 = pl.pallas_call(
        _gcn_body,
        grid=(B // Bb,),
        in_specs=[
            pl.BlockSpec((_N, Bb, 256), lambda i: (0, i, 0)),
            pl.BlockSpec((256, 512), lambda i: (0, 0)),
            pl.BlockSpec((1, 256), lambda i: (0, 0)),
            pl.BlockSpec((256, 128), lambda i: (0, 0)),
            pl.BlockSpec((1, 64), lambda i: (0, 0)),
            pl.BlockSpec((64, 6), lambda i: (0, 0)),
            pl.BlockSpec((1, 3), lambda i: (0, 0)),
        ],
        out_specs=pl.BlockSpec((Bb, _N * 3), lambda i: (i, 0)),
        out_shape=jax.ShapeDtypeStruct((B, _N * 3), jnp.float32),
    )(xt, wc1, b1.reshape(1, 256), wc2, b2.reshape(1, 64),
      wc3, b3.reshape(1, 3))
    return out2.reshape(B, _N, 3)
